# Initial kernel scaffold; baseline (speedup 1.0000x reference)
import jax
import jax.numpy as jnp
from jax.experimental import pallas as pl


def kernel(x_user, x_group, x_item, ei_ui_src, ei_ui_dst, ei_ug_src, ei_ug_dst, params):
    raise NotImplementedError("write your pallas kernel here")


# speedup vs baseline: 9.3933x; 9.3933x over previous
import functools

import jax
import jax.numpy as jnp
from jax import lax
from jax.experimental import pallas as pl
from jax.experimental.pallas import tpu as pltpu
from jax.experimental.pallas import tpu_sc as plsc

D = 128
NSC = 2
NTILE = 16
LANES = 16
CH = 128
ROWW = 80


def _mm_bias_body(x_ref, w_ref, b_ref, o_ref):
    o_ref[...] = jnp.dot(x_ref[...], w_ref[...],
                         preferred_element_type=jnp.float32) + b_ref[...]


def _mm_bias(x, w, b, bm):
    n = x.shape[0]
    return pl.pallas_call(
        _mm_bias_body,
        grid=(n // bm,),
        in_specs=[pl.BlockSpec((bm, D), lambda i: (i, 0)),
                  pl.BlockSpec((D, D), lambda i: (0, 0)),
                  pl.BlockSpec((1, D), lambda i: (0, 0))],
        out_specs=pl.BlockSpec((bm, D), lambda i: (i, 0)),
        out_shape=jax.ShapeDtypeStruct((n, D), jnp.float32),
    )(x, w, b)


def _split_w(w):
    return w.reshape(D, 2, 64).transpose(1, 0, 2)


def _mm_split_body(x_ref, w_ref, o_ref):
    o_ref[...] = jnp.dot(x_ref[...], w_ref[0],
                         preferred_element_type=jnp.float32)[None]


def _mm_split(x, w, bm):
    n = x.shape[0]
    return pl.pallas_call(
        _mm_split_body,
        grid=(2, n // bm),
        in_specs=[pl.BlockSpec((bm, D), lambda j, i: (i, 0)),
                  pl.BlockSpec((1, D, 64), lambda j, i: (j, 0, 0))],
        out_specs=pl.BlockSpec((1, bm, 64), lambda j, i: (j, i, 0)),
        out_shape=jax.ShapeDtypeStruct((2, n, 64), jnp.float32),
    )(x, _split_w(w))


def _mm_aux_body(x_ref, v_ref, o_ref):
    o_ref[...] = jnp.dot(x_ref[...], v_ref[...],
                         preferred_element_type=jnp.float32)


def _mm_aux(x, v, bm):
    n = x.shape[0]
    return pl.pallas_call(
        _mm_aux_body,
        grid=(n // bm,),
        in_specs=[pl.BlockSpec((bm, D), lambda i: (i, 0)),
                  pl.BlockSpec((D, 8), lambda i: (0, 0))],
        out_specs=pl.BlockSpec((bm, 8), lambda i: (i, 0)),
        out_shape=jax.ShapeDtypeStruct((n, 8), jnp.float32),
    )(x, v)


def _relu_mm_split_body(pre_ref, feat_ref, den_ref, w_ref, o_ref):
    h = jnp.maximum(pre_ref[...] + feat_ref[...] / (den_ref[...] + 1e-16), 0.0)
    o_ref[...] = jnp.dot(h, w_ref[0], preferred_element_type=jnp.float32)[None]


def _relu_mm_split(pre, feat, den, w, bm):
    n = pre.shape[0]
    return pl.pallas_call(
        _relu_mm_split_body,
        grid=(2, n // bm),
        in_specs=[pl.BlockSpec((bm, D), lambda j, i: (i, 0)),
                  pl.BlockSpec((bm, D), lambda j, i: (i, 0)),
                  pl.BlockSpec((bm, 1), lambda j, i: (i, 0)),
                  pl.BlockSpec((1, D, 64), lambda j, i: (j, 0, 0))],
        out_specs=pl.BlockSpec((1, bm, 64), lambda j, i: (j, i, 0)),
        out_shape=jax.ShapeDtypeStruct((2, n, 64), jnp.float32),
    )(pre, feat, den, _split_w(w))


def _relu_mm_aux_body(pre_ref, feat_ref, den_ref, w_ref, b_ref, v_ref,
                      o_ref, aux_ref):
    h = jnp.maximum(pre_ref[...] + feat_ref[...] / (den_ref[...] + 1e-16), 0.0)
    o_ref[...] = jnp.dot(h, w_ref[...],
                         preferred_element_type=jnp.float32) + b_ref[...]
    aux_ref[...] = jnp.dot(h, v_ref[...], preferred_element_type=jnp.float32)


def _relu_mm_aux(pre, feat, den, w, b, v):
    n = pre.shape[0]
    return pl.pallas_call(
        _relu_mm_aux_body,
        grid=(1,),
        in_specs=[pl.BlockSpec((n, D), lambda i: (0, 0)),
                  pl.BlockSpec((n, D), lambda i: (0, 0)),
                  pl.BlockSpec((n, 1), lambda i: (0, 0)),
                  pl.BlockSpec((D, D), lambda i: (0, 0)),
                  pl.BlockSpec((1, D), lambda i: (0, 0)),
                  pl.BlockSpec((D, 8), lambda i: (0, 0))],
        out_specs=[pl.BlockSpec((n, D), lambda i: (0, 0)),
                   pl.BlockSpec((n, 8), lambda i: (0, 0))],
        out_shape=[jax.ShapeDtypeStruct((n, D), jnp.float32),
                   jax.ShapeDtypeStruct((n, 8), jnp.float32)],
    )(pre, feat, den, w, b, v)


def _pred_body(pre_ref, feat_ref, den_ref, w_ref, b_ref, o_ref):
    h = jnp.maximum(pre_ref[...] + feat_ref[...] / (den_ref[...] + 1e-16), 0.0)
    o_ref[...] = jnp.dot(h, w_ref[...],
                         preferred_element_type=jnp.float32) + b_ref[...]


def _pred(pre, feat, den, w, b, bm):
    n, m = pre.shape[0], w.shape[1]
    return pl.pallas_call(
        _pred_body,
        grid=(n // bm,),
        in_specs=[pl.BlockSpec((bm, D), lambda i: (i, 0)),
                  pl.BlockSpec((bm, D), lambda i: (i, 0)),
                  pl.BlockSpec((bm, 1), lambda i: (i, 0)),
                  pl.BlockSpec((D, m), lambda i: (0, 0)),
                  pl.BlockSpec((1, m), lambda i: (0, 0))],
        out_specs=pl.BlockSpec((bm, m), lambda i: (i, 0)),
        out_shape=jax.ShapeDtypeStruct((n, m), jnp.float32),
    )(pre, feat, den, w, b)


def _gather_rows(rows_hbm, idx_ref, out_ref, sem):
    pltpu.async_copy(rows_hbm.at[idx_ref], out_ref, sem).wait()


def _scatter_add_rows(rows_ref, acc_ref, idx_ref):
    pltpu.sync_copy(rows_ref, acc_ref.at[idx_ref], add=True)

_fori = lax.fori_loop
_axis = lax.axis_index


def _make_sc_edge_body(groups, bufs, rounds):
    ng = len(groups)

    scratch = []
    for p in groups:
        scratch.append(pltpu.VMEM((p["n_src"],), jnp.float32))
        if p["has_d"]:
            scratch.append(pltpu.VMEM((p["n_dst"],), jnp.float32))
    scratch += [
        pltpu.VMEM((ng, LANES), jnp.float32),
        pltpu.VMEM((CH,), jnp.int32),
        pltpu.VMEM((CH,), jnp.int32),
        pltpu.VMEM((CH,), jnp.int32),
        pltpu.VMEM((CH,), jnp.float32),
        pltpu.VMEM((CH, 64), jnp.float32),
        pltpu.VMEM((CH, ROWW), jnp.float32),
    ]
    scratch += [pltpu.VMEM_SHARED((r, ROWW), jnp.float32) for (r, _) in
                [(b[0], None) for b in bufs]]
    scratch += [pltpu.SemaphoreType.DMA]

    out_type = [jax.ShapeDtypeStruct((2 * p["n_dst"], ROWW), jnp.float32)
                for p in groups]

    def body(*refs):
        n_in = 4 * ng + sum(p["has_d"] for p in groups) + 1
        ins = refs[:n_in]
        outs = refs[n_in:n_in + ng]
        scr = refs[n_in + ng:]

        pin, k = [], 0
        for p in groups:
            e = {"src": ins[k], "dst": ins[k + 1], "s": ins[k + 2]}
            k += 3
            if p["has_d"]:
                e["d"] = ins[k]; k += 1
            e["rows"] = ins[k]; k += 1
            pin.append(e)
        misc_in = ins[k]

        sk = 0
        for i, p in enumerate(groups):
            pin[i]["s_v"] = scr[sk]; sk += 1
            if p["has_d"]:
                pin[i]["d_v"] = scr[sk]; sk += 1
        misc_v, src_v, dst_v, idx2_v, w_v, gath_v, rows_v = scr[sk:sk + 7]
        sk += 7
        accs = scr[sk:sk + len(bufs)]
        sem = scr[sk + len(bufs)]

        cid = _axis("c")
        tid = _axis("s")

        pltpu.sync_copy(misc_in, misc_v)
        for i, p in enumerate(groups):
            pltpu.sync_copy(pin[i]["s"], pin[i]["s_v"])
            if p["has_d"]:
                pltpu.sync_copy(pin[i]["d"], pin[i]["d_v"])

        zero16 = jnp.zeros((LANES,), jnp.float32)

        def _zero_rows_v():
            def zrow(i, c):
                for q in range(ROWW // LANES):
                    rows_v[i, pl.ds(LANES * q, LANES)] = zero16
                return c
            _fori(0, CH, zrow, 0)

        ZB = 80

        def run_pass(p, e, acc_sp, m_idx, dbase, nd):
            m16 = misc_v[m_idx]
            n_src, e_real, cpt = p["n_src"], p["e_real"], p["cpt"]
            has_d = p["has_d"]

            def chunk_body(c, carry):
                base = (tid * cpt + c) * CH
                pltpu.sync_copy(e["src"].at[pl.ds(base, CH)], src_v)
                pltpu.sync_copy(e["dst"].at[pl.ds(base, CH)], dst_v)

                def lane_body(kk, cc):
                    sl = pl.ds(LANES * kk, LANES)
                    sidx = src_v[sl]
                    didx = dst_v[sl]
                    lg = plsc.load_gather(e["s_v"], [sidx])
                    if has_d:
                        lg = lg + plsc.load_gather(e["d_v"], [didx])
                    lg = jnp.maximum(lg, 0.2 * lg)
                    w = jnp.exp(lg - m16)
                    g = base + LANES * kk + lax.iota(jnp.int32, LANES)
                    valid = g < e_real
                    if nd != p["n_dst"]:
                        valid = valid & (didx >= dbase) & (didx < dbase + nd)
                        dst_v[sl] = jnp.where(valid, didx - dbase, 0)
                    w_v[sl] = jnp.where(valid, w, 0.0)
                    idx2_v[sl] = sidx + cid * n_src
                    return cc
                _fori(0, CH // LANES, lane_body, 0)

                _gather_rows(e["rows"], idx2_v, gath_v, sem)

                lane0 = lax.iota(jnp.int32, LANES) == 0

                def scale_body(j, cc):
                    wb = plsc.load_gather(
                        w_v, [jnp.zeros((LANES,), jnp.int32) + j])
                    for q in range(4):
                        sl = pl.ds(LANES * q, LANES)
                        rows_v[j, sl] = gath_v[j, sl] * wb
                    rows_v[j, pl.ds(64, LANES)] = jnp.where(lane0, wb, 0.0)
                    return cc
                _fori(0, CH, scale_body, 0)

                _scatter_add_rows(rows_v, acc_sp, dst_v)
                return carry
            _fori(0, cpt, chunk_body, 0)

        for rnd in rounds:
            _zero_rows_v()
            for (gi, bi, dbase) in rnd:
                nblocks = bufs[bi][0] // ZB
                for j in range(-(-nblocks // NTILE)):
                    m = (j * NTILE + tid) % nblocks
                    pltpu.sync_copy(rows_v.at[pl.ds(0, ZB)],
                                    accs[bi].at[pl.ds(m * ZB, ZB)])
            plsc.subcore_barrier()
            for (gi, bi, dbase) in rnd:
                run_pass(groups[gi], pin[gi], accs[bi], gi, dbase,
                         bufs[bi][0])
            plsc.subcore_barrier()
            for (gi, bi, dbase) in rnd:
                nd_tot = groups[gi]["n_dst"]
                nblocks = bufs[bi][0] // ZB
                for j in range(-(-nblocks // NTILE)):
                    b = ((j * NTILE + tid) % nblocks) * ZB
                    pltpu.sync_copy(
                        accs[bi].at[pl.ds(b, ZB)],
                        outs[gi].at[pl.ds(cid * nd_tot + dbase + b, ZB)])
            plsc.subcore_barrier()

    return body, out_type, scratch


def _make_sc_edge_kernel(groups, bufs, rounds):
    body, out_type, scratch = _make_sc_edge_body(groups, bufs, rounds)
    mesh = plsc.VectorSubcoreMesh(core_axis_name="c", subcore_axis_name="s",
                                  num_cores=NSC, num_subcores=NTILE)
    return pl.kernel(body, out_type=out_type, mesh=mesh,
                     scratch_types=scratch,
                     compiler_params=pltpu.CompilerParams(
                         needs_layout_passes=False,
                         use_tc_tiling_on_sc=False))


_EDGE_K1 = None
_EDGE_K2 = None


def _get_edge_kernels():
    global _EDGE_K1, _EDGE_K2
    if _EDGE_K1 is None:
        _EDGE_K1 = _make_sc_edge_kernel(
            groups=[
                dict(n_src=10000, n_dst=20000, e_real=200000,
                     cpt=200704 // (NTILE * CH), has_d=True),
                dict(n_src=20000, n_dst=2000, e_real=100000,
                     cpt=100352 // (NTILE * CH), has_d=False),
            ],
            bufs=[(10000, ROWW), (2000, ROWW)],
            rounds=[[(0, 0, 0), (1, 1, 0)], [(0, 0, 10000)]],
        )
        _EDGE_K2 = _make_sc_edge_kernel(
            groups=[dict(n_src=20000, n_dst=2000, e_real=100000,
                         cpt=100352 // (NTILE * CH), has_d=True)],
            bufs=[(2000, ROWW)],
            rounds=[[(0, 0, 0)]],
        )
    return _EDGE_K1, _EDGE_K2


def _leaky(x):
    return jnp.maximum(x, 0.2 * x)


def kernel(x_user, x_group, x_item, ei_ui_src, ei_ui_dst, ei_ug_src,
           ei_ug_dst, params):
    ek1, ek2 = _get_edge_kernels()
    U = params["emb"]["user"]
    I = params["emb"]["item"]
    L1, L2 = params["layer1"], params["layer2"]
    c_iu, c_ug1 = L1["conv"]["iu"], L1["conv"]["ug"]
    c_gu1, c_ug2 = L1["conv"]["gu"], L2["conv"]["ug"]

    v_d_iu = c_iu["Wd"] @ c_iu["a_d"]
    v_s_ug1 = c_ug1["Ws"] @ c_ug1["a_s"]
    v_s_iu = c_iu["Ws"] @ c_iu["a_s"]
    v_s_ug2 = c_ug2["Ws"] @ c_ug2["a_s"]
    v_d_ug2 = c_ug2["Wd"] @ c_ug2["a_d"]
    pad8 = lambda *vs: jnp.stack(list(vs) + [jnp.zeros((D,), jnp.float32)] *
                                 (8 - len(vs)), axis=1)

    b1f = (L1["lin"]["user"]["b"] + c_iu["b"] + c_gu1["b"])[None]
    bias1g = (L1["lin"]["group"]["b"] + c_ug1["b"])[None]
    b2gf = (L2["lin"]["group"]["b"] + c_ug2["b"])[None]

    u_pre = _mm_bias(U, L1["lin"]["user"]["W"], b1f, 2000)
    Bstack = _mm_split(U, c_ug1["Ws"], 2000).reshape(2 * 20000, 64)
    Astack = _mm_split(I, c_iu["Ws"], 2000).reshape(2 * 10000, 64)
    aux_u = _mm_aux(U, pad8(v_d_iu, v_s_ug1), 2000)
    aux_i = _mm_aux(I, pad8(v_s_iu), 2000)
    d_iu, s_ug1, s_iu = aux_u[:, 0], aux_u[:, 1], aux_i[:, 0]

    m_iu = _leaky(jnp.max(s_iu) + jnp.max(d_iu))
    m_ug1 = _leaky(jnp.max(s_ug1))
    misc1 = jnp.stack([jnp.full((LANES,), m_iu, jnp.float32),
                       jnp.full((LANES,), m_ug1, jnp.float32)])

    src_iu = jnp.pad(ei_ui_dst, (0, 200704 - 200000))
    dst_iu = jnp.pad(ei_ui_src, (0, 200704 - 200000))
    src_ug = jnp.pad(ei_ug_src, (0, 100352 - 100000))
    dst_ug = jnp.pad(ei_ug_dst, (0, 100352 - 100000))

    acc_u, acc_g = ek1(src_iu, dst_iu, s_iu, d_iu, Astack,
                       src_ug, dst_ug, s_ug1, Bstack, misc1)

    feat_u = jnp.concatenate([acc_u[:20000, :64], acc_u[20000:, :64]], axis=1)
    den_u = acc_u[:20000, 64:65]
    feat_g = jnp.concatenate([acc_g[:2000, :64], acc_g[2000:, :64]], axis=1)
    den_g = acc_g[:2000, 64:65]

    Cstack = _relu_mm_split(u_pre, feat_u, den_u, c_ug2["Ws"], 2000)
    s_ug2 = (Cstack[0] @ c_ug2["a_s"][:64] + Cstack[1] @ c_ug2["a_s"][64:])
    Cstack = Cstack.reshape(2 * 20000, 64)

    pre_g = jnp.broadcast_to(bias1g, (2000, D))
    g_pre2, aux_g2 = _relu_mm_aux(pre_g, feat_g, den_g,
                                  L2["lin"]["group"]["W"], b2gf,
                                  pad8(v_d_ug2))
    d_ug2 = aux_g2[:, 0]

    m2 = _leaky(jnp.max(s_ug2) + jnp.max(d_ug2))
    misc2 = jnp.full((1, LANES), m2, jnp.float32)

    (acc2,) = ek2(src_ug, dst_ug, s_ug2, d_ug2, Cstack, misc2)
    feat2 = jnp.concatenate([acc2[:2000, :64], acc2[2000:, :64]], axis=1)
    den2 = acc2[:2000, 64:65]

    return _pred(g_pre2, feat2, den2, params["pred"]["W"],
                 params["pred"]["b"][None], 400)
